# baseline (speedup 1.0000x reference)
import functools

import jax
import jax.numpy as jnp
from jax import lax
from jax.experimental import pallas as pl
from jax.experimental.pallas import tpu as pltpu
from jax.experimental.pallas import tpu_sc as plsc

BETA = 0.25

_TM = 512
_TK = 1024

_NC = 2
_NS = 16
_NW = _NC * _NS
_CHUNK = 128


def _argmin_body(z_ref, w_ref, idx_ref, loss_ref, min_s, idx_s, zsq_s,
                 *, nk, ni, scale):
    i = pl.program_id(0)
    k = pl.program_id(1)
    z = z_ref[...]
    w = w_ref[...]

    @pl.when(k == 0)
    def _init():
        zsq_s[...] = jnp.sum(z * z, axis=1, keepdims=True)
        min_s[...] = jnp.full(min_s.shape, jnp.inf, jnp.float32)
        idx_s[...] = jnp.zeros(idx_s.shape, jnp.int32)

    dot = lax.dot_general(z, w, (((1,), (1,)), ((), ())),
                          preferred_element_type=jnp.float32)
    wsq = jnp.sum(w * w, axis=1)
    d = (zsq_s[...] - 2.0 * dot) + wsq[None, :]
    tmin = jnp.min(d, axis=1, keepdims=True)
    jidx = lax.broadcasted_iota(jnp.int32, d.shape, 1) + k * _TK
    cand = jnp.where(d == tmin, jidx, jnp.int32(2**30))
    tidx = jnp.min(cand, axis=1, keepdims=True)
    upd = tmin < min_s[...]
    min_s[...] = jnp.where(upd, tmin, min_s[...])
    idx_s[...] = jnp.where(upd, tidx, idx_s[...])

    @pl.when(k == nk - 1)
    def _emit():
        idx_ref[...] = idx_s[...][None]
        part = jnp.sum(min_s[...], keepdims=True)
        prev = jnp.where(i == 0, jnp.zeros_like(loss_ref[...]), loss_ref[...])
        tot = prev + part
        loss_ref[...] = jnp.where(i == ni - 1, tot * scale, tot)


def _vq_argmin(zf, w):
    m, d_dim = zf.shape
    k_tot = w.shape[0]
    ni = m // _TM
    nk = k_tot // _TK
    scale = (1.0 + BETA) / (m * d_dim)
    idx3, loss = pl.pallas_call(
        functools.partial(_argmin_body, nk=nk, ni=ni, scale=scale),
        grid=(ni, nk),
        in_specs=[
            pl.BlockSpec((_TM, d_dim), lambda i, k: (i, 0)),
            pl.BlockSpec((_TK, d_dim), lambda i, k: (k, 0)),
        ],
        out_specs=[
            pl.BlockSpec((1, _TM, 1), lambda i, k: (i, 0, 0)),
            pl.BlockSpec((1, 1), lambda i, k: (0, 0)),
        ],
        out_shape=[
            jax.ShapeDtypeStruct((ni, _TM, 1), jnp.int32),
            jax.ShapeDtypeStruct((1, 1), jnp.float32),
        ],
        scratch_shapes=[
            pltpu.VMEM((_TM, 1), jnp.float32),
            pltpu.VMEM((_TM, 1), jnp.int32),
            pltpu.VMEM((_TM, 1), jnp.float32),
        ],
        compiler_params=pltpu.CompilerParams(
            dimension_semantics=("arbitrary", "arbitrary")),
    )(zf, w)
    return idx3.reshape(m), loss[0, 0]


def _gather_body(w_hbm, idx_hbm, out_hbm, idx_v, rows_v, sem):
    wid = lax.axis_index("s") * _NC + lax.axis_index("c")
    b_per_w = idx_hbm.shape[0] // _NW
    base = wid * b_per_w
    nch = b_per_w // _CHUNK

    def chunk(c, carry):
        off = base + c * _CHUNK
        pltpu.sync_copy(idx_hbm.at[pl.ds(off, _CHUNK)], idx_v)
        pltpu.async_copy(w_hbm.at[idx_v], rows_v, sem).wait()
        pltpu.sync_copy(rows_v, out_hbm.at[pl.ds(off, _CHUNK)])
        return carry

    lax.fori_loop(0, nch, chunk, 0)


def _vq_gather(w, idx):
    m = idx.shape[0]
    d_dim = w.shape[1]
    mesh = plsc.VectorSubcoreMesh(core_axis_name="c", subcore_axis_name="s")
    fn = functools.partial(
        pl.kernel,
        mesh=mesh,
        out_type=jax.ShapeDtypeStruct((m, d_dim), jnp.float32),
        scratch_types=[
            pltpu.VMEM((_CHUNK,), jnp.int32),
            pltpu.VMEM((_CHUNK, d_dim), jnp.float32),
            pltpu.SemaphoreType.DMA,
        ],
    )(_gather_body)
    return fn(w, idx)


def kernel(z, W):
    zf = z.reshape(-1, z.shape[-1])
    idx, loss = _vq_argmin(zf, W)
    z_q = _vq_gather(W, idx)
    z_q_st = z_q.reshape(z.shape)
    min_encoding_indices = idx.reshape(z.shape[:-1] + (1,))
    return (z_q_st, loss, min_encoding_indices)


# speedup vs baseline: 1.0872x; 1.0872x over previous
import functools

import numpy as np

import jax
import jax.numpy as jnp
from jax import lax
from jax.experimental import pallas as pl
from jax.experimental.pallas import tpu as pltpu
from jax.experimental.pallas import tpu_sc as plsc

BETA = 0.25

_TM = 512
_TK = 1024

_NC = 2
_NS = 16
_NW = _NC * _NS
_CHUNK = 128

_IOTA_ROW = np.arange(_TK, dtype=np.float32)[None, :]


def _argmin_body(z_ref, w_ref, iota_ref, idx_ref, loss_ref,
                 min_s, idx_s, zsq_s, wsq_s, *, nk, ni, scale):
    i = pl.program_id(0)
    k = pl.program_id(1)
    z = z_ref[...]

    @pl.when(k == 0)
    def _init():
        zsq_s[...] = jnp.sum(z * z, axis=1, keepdims=True)
        min_s[...] = jnp.full(min_s.shape, jnp.inf, jnp.float32)
        idx_s[...] = jnp.zeros(idx_s.shape, jnp.float32)

    @pl.when(i == 0)
    def _wsq():
        w = w_ref[...]
        wsq_s[pl.ds(k, 1), :] = jnp.sum(w * w, axis=1)[None, :]

    dot = lax.dot_general(z, w_ref[...], (((1,), (1,)), ((), ())),
                          preferred_element_type=jnp.float32)
    d = (zsq_s[...] - 2.0 * dot) + wsq_s[pl.ds(k, 1), :]
    tmin = jnp.min(d, axis=1, keepdims=True)
    jidx = iota_ref[...]
    cand = jnp.where(d == tmin, jidx, jnp.float32(3.0e38))
    tidx = (jnp.min(cand, axis=1, keepdims=True)
            + (k * _TK).astype(jnp.float32))
    upd = tmin < min_s[...]
    min_s[...] = jnp.where(upd, tmin, min_s[...])
    idx_s[...] = jnp.where(upd, tidx, idx_s[...])

    @pl.when(k == nk - 1)
    def _emit():
        idx_ref[...] = idx_s[...].astype(jnp.int32)[None]
        part = jnp.sum(min_s[...], keepdims=True)
        prev = jnp.where(i == 0, jnp.zeros_like(loss_ref[...]), loss_ref[...])
        tot = prev + part
        loss_ref[...] = jnp.where(i == ni - 1, tot * scale, tot)


def _vq_argmin(zf, w):
    m, d_dim = zf.shape
    k_tot = w.shape[0]
    ni = m // _TM
    nk = k_tot // _TK
    scale = (1.0 + BETA) / (m * d_dim)
    idx3, loss = pl.pallas_call(
        functools.partial(_argmin_body, nk=nk, ni=ni, scale=scale),
        grid=(ni, nk),
        in_specs=[
            pl.BlockSpec((_TM, d_dim), lambda i, k: (i, 0)),
            pl.BlockSpec((_TK, d_dim), lambda i, k: (k, 0)),
            pl.BlockSpec((1, _TK), lambda i, k: (0, 0)),
        ],
        out_specs=[
            pl.BlockSpec((1, _TM, 1), lambda i, k: (i, 0, 0)),
            pl.BlockSpec((1, 1), lambda i, k: (0, 0)),
        ],
        out_shape=[
            jax.ShapeDtypeStruct((ni, _TM, 1), jnp.int32),
            jax.ShapeDtypeStruct((1, 1), jnp.float32),
        ],
        scratch_shapes=[
            pltpu.VMEM((_TM, 1), jnp.float32),
            pltpu.VMEM((_TM, 1), jnp.float32),
            pltpu.VMEM((_TM, 1), jnp.float32),
            pltpu.VMEM((nk, _TK), jnp.float32),
        ],
        compiler_params=pltpu.CompilerParams(
            dimension_semantics=("arbitrary", "arbitrary")),
    )(zf, w, jnp.asarray(_IOTA_ROW))
    return idx3.reshape(m), loss[0, 0]


def _gather_body(w_hbm, idx_hbm, out_hbm, idx_v, rows_v, sem):
    wid = lax.axis_index("s") * _NC + lax.axis_index("c")
    b_per_w = idx_hbm.shape[0] // _NW
    base = wid * b_per_w
    nch = b_per_w // _CHUNK

    def chunk(c, carry):
        off = base + c * _CHUNK
        pltpu.sync_copy(idx_hbm.at[pl.ds(off, _CHUNK)], idx_v)
        pltpu.async_copy(w_hbm.at[idx_v], rows_v, sem).wait()
        pltpu.sync_copy(rows_v, out_hbm.at[pl.ds(off, _CHUNK)])
        return carry

    lax.fori_loop(0, nch, chunk, 0)


def _vq_gather(w, idx):
    m = idx.shape[0]
    d_dim = w.shape[1]
    mesh = plsc.VectorSubcoreMesh(core_axis_name="c", subcore_axis_name="s")
    fn = functools.partial(
        pl.kernel,
        mesh=mesh,
        out_type=jax.ShapeDtypeStruct((m, d_dim), jnp.float32),
        scratch_types=[
            pltpu.VMEM((_CHUNK,), jnp.int32),
            pltpu.VMEM((_CHUNK, d_dim), jnp.float32),
            pltpu.SemaphoreType.DMA,
        ],
    )(_gather_body)
    return fn(w, idx)


def kernel(z, W):
    zf = z.reshape(-1, z.shape[-1])
    idx, loss = _vq_argmin(zf, W)
    z_q = _vq_gather(W, idx)
    z_q_st = z_q.reshape(z.shape)
    min_encoding_indices = idx.reshape(z.shape[:-1] + (1,))
    return (z_q_st, loss, min_encoding_indices)


# speedup vs baseline: 1.1091x; 1.0201x over previous
import functools

import numpy as np

import jax
import jax.numpy as jnp
from jax import lax
from jax.experimental import pallas as pl
from jax.experimental.pallas import tpu as pltpu
from jax.experimental.pallas import tpu_sc as plsc

BETA = 0.25

_TM = 512
_TK = 1024

_NC = 2
_NS = 16
_NW = _NC * _NS
_CHUNK = 128

_IOTA_COL = np.arange(_TK, dtype=np.float32)[:, None]


def _argmin_body(z_ref, w_ref, iota_ref, idx_ref, loss_ref,
                 min_s, idx_s, zsq_s, wsq_s, *, nk, ni, scale):
    i = pl.program_id(0)
    k = pl.program_id(1)
    z = z_ref[...]
    w = w_ref[...]

    @pl.when(k == 0)
    def _init():
        u = z * z
        ones = jnp.ones((1, u.shape[1]), jnp.float32)
        zsq_s[...] = lax.dot_general(ones, u, (((1,), (1,)), ((), ())),
                                     preferred_element_type=jnp.float32)
        min_s[...] = jnp.full(min_s.shape, jnp.inf, jnp.float32)
        idx_s[...] = jnp.zeros(idx_s.shape, jnp.float32)

    @pl.when(i == 0)
    def _wsq():
        wsq_s[pl.ds(k * _TK, _TK), :] = jnp.sum(w * w, axis=1, keepdims=True)

    dot = lax.dot_general(w, z, (((1,), (1,)), ((), ())),
                          preferred_element_type=jnp.float32)
    d = (zsq_s[...] - 2.0 * dot) + wsq_s[pl.ds(k * _TK, _TK), :]
    tmin = jnp.min(d, axis=0, keepdims=True)
    jidx = iota_ref[...]
    cand = jnp.where(d == tmin, jidx, jnp.float32(3.0e38))
    tidx = (jnp.min(cand, axis=0, keepdims=True)
            + (k * _TK).astype(jnp.float32))
    upd = tmin < min_s[...]
    min_s[...] = jnp.where(upd, tmin, min_s[...])
    idx_s[...] = jnp.where(upd, tidx, idx_s[...])

    @pl.when(k == nk - 1)
    def _emit():
        idx_ref[...] = idx_s[...].astype(jnp.int32)[None]
        part = jnp.sum(min_s[...], keepdims=True)
        prev = jnp.where(i == 0, jnp.zeros_like(loss_ref[...]), loss_ref[...])
        tot = prev + part
        loss_ref[...] = jnp.where(i == ni - 1, tot * scale, tot)


def _vq_argmin(zf, w):
    m, d_dim = zf.shape
    k_tot = w.shape[0]
    ni = m // _TM
    nk = k_tot // _TK
    scale = (1.0 + BETA) / (m * d_dim)
    idx3, loss = pl.pallas_call(
        functools.partial(_argmin_body, nk=nk, ni=ni, scale=scale),
        grid=(ni, nk),
        in_specs=[
            pl.BlockSpec((_TM, d_dim), lambda i, k: (i, 0)),
            pl.BlockSpec((_TK, d_dim), lambda i, k: (k, 0)),
            pl.BlockSpec((_TK, 1), lambda i, k: (0, 0)),
        ],
        out_specs=[
            pl.BlockSpec((1, 1, _TM), lambda i, k: (i, 0, 0)),
            pl.BlockSpec((1, 1), lambda i, k: (0, 0)),
        ],
        out_shape=[
            jax.ShapeDtypeStruct((ni, 1, _TM), jnp.int32),
            jax.ShapeDtypeStruct((1, 1), jnp.float32),
        ],
        scratch_shapes=[
            pltpu.VMEM((1, _TM), jnp.float32),
            pltpu.VMEM((1, _TM), jnp.float32),
            pltpu.VMEM((1, _TM), jnp.float32),
            pltpu.VMEM((nk * _TK, 1), jnp.float32),
        ],
        compiler_params=pltpu.CompilerParams(
            dimension_semantics=("arbitrary", "arbitrary")),
    )(zf, w, jnp.asarray(_IOTA_COL))
    return idx3.reshape(m), loss[0, 0]


def _gather_body(w_hbm, idx_hbm, out_hbm, idx_v, rows_v, sem):
    wid = lax.axis_index("s") * _NC + lax.axis_index("c")
    b_per_w = idx_hbm.shape[0] // _NW
    base = wid * b_per_w
    nch = b_per_w // _CHUNK

    def chunk(c, carry):
        off = base + c * _CHUNK
        pltpu.sync_copy(idx_hbm.at[pl.ds(off, _CHUNK)], idx_v)
        pltpu.async_copy(w_hbm.at[idx_v], rows_v, sem).wait()
        pltpu.sync_copy(rows_v, out_hbm.at[pl.ds(off, _CHUNK)])
        return carry

    lax.fori_loop(0, nch, chunk, 0)


def _vq_gather(w, idx):
    m = idx.shape[0]
    d_dim = w.shape[1]
    mesh = plsc.VectorSubcoreMesh(core_axis_name="c", subcore_axis_name="s")
    fn = functools.partial(
        pl.kernel,
        mesh=mesh,
        out_type=jax.ShapeDtypeStruct((m, d_dim), jnp.float32),
        scratch_types=[
            pltpu.VMEM((_CHUNK,), jnp.int32),
            pltpu.VMEM((_CHUNK, d_dim), jnp.float32),
            pltpu.SemaphoreType.DMA,
        ],
    )(_gather_body)
    return fn(w, idx)


def kernel(z, W):
    zf = z.reshape(-1, z.shape[-1])
    idx, loss = _vq_argmin(zf, W)
    z_q = _vq_gather(W, idx)
    z_q_st = z_q.reshape(z.shape)
    min_encoding_indices = idx.reshape(z.shape[:-1] + (1,))
    return (z_q_st, loss, min_encoding_indices)


# speedup vs baseline: 1.1920x; 1.0747x over previous
import functools

import numpy as np

import jax
import jax.numpy as jnp
from jax import lax
from jax.experimental import pallas as pl
from jax.experimental.pallas import tpu as pltpu
from jax.experimental.pallas import tpu_sc as plsc

BETA = 0.25

_TM = 512
_TK = 1024

_NC = 2
_NS = 16
_NW = _NC * _NS
_CHUNK = 128

_IOTA_COL = np.arange(_TK, dtype=np.float32)[:, None]


def _argmin_body(z_ref, w_ref, idx_ref, loss_ref,
                 acc_v, acc_j, zsq_s, wsq_s, *, nk, ni, scale):
    i = pl.program_id(0)
    k = pl.program_id(1)
    z = z_ref[...]
    w = w_ref[...]
    tm = z.shape[0]

    @pl.when(k == 0)
    def _init():
        u = z * z
        ones = jnp.ones((1, u.shape[1]), jnp.float32)
        zsq_s[...] = lax.dot_general(ones, u, (((1,), (1,)), ((), ())),
                                     preferred_element_type=jnp.float32)

    @pl.when(i == 0)
    def _wsq():
        wsq_s[pl.ds(k * _TK, _TK), :] = jnp.sum(w * w, axis=1, keepdims=True)

    dot = lax.dot_general(w, z, (((1,), (1,)), ((), ())),
                          preferred_element_type=jnp.float32)
    zrow = zsq_s[...]
    wcol = wsq_s[pl.ds(k * _TK, _TK), :]

    av = jnp.where(k == 0, jnp.full((8, tm), jnp.inf, jnp.float32), acc_v[...])
    aj = jnp.where(k == 0, jnp.zeros((8, tm), jnp.float32), acc_j[...])
    svec = (lax.broadcasted_iota(jnp.int32, (8, tm), 0).astype(jnp.float32)
            + (k * _TK).astype(jnp.float32))
    for r in range(_TK // 8):
        blk = (zrow - 2.0 * dot[r * 8:(r + 1) * 8, :]) + wcol[r * 8:(r + 1) * 8, :]
        m = blk < av
        av = jnp.where(m, blk, av)
        aj = jnp.where(m, svec + jnp.float32(8 * r), aj)
    acc_v[...] = av
    acc_j[...] = aj

    @pl.when(k == nk - 1)
    def _emit():
        tmin = jnp.min(av, axis=0, keepdims=True)
        jrow = jnp.min(jnp.where(av == tmin, aj, jnp.float32(3.0e38)),
                       axis=0, keepdims=True)
        idx_ref[...] = jrow.astype(jnp.int32)[None]
        part = jnp.sum(tmin, keepdims=True)
        prev = jnp.where(i == 0, jnp.zeros_like(loss_ref[...]), loss_ref[...])
        tot = prev + part
        loss_ref[...] = jnp.where(i == ni - 1, tot * scale, tot)


def _vq_argmin(zf, w):
    m, d_dim = zf.shape
    k_tot = w.shape[0]
    ni = m // _TM
    nk = k_tot // _TK
    scale = (1.0 + BETA) / (m * d_dim)
    idx3, loss = pl.pallas_call(
        functools.partial(_argmin_body, nk=nk, ni=ni, scale=scale),
        grid=(ni, nk),
        in_specs=[
            pl.BlockSpec((_TM, d_dim), lambda i, k: (i, 0)),
            pl.BlockSpec((_TK, d_dim), lambda i, k: (k, 0)),
        ],
        out_specs=[
            pl.BlockSpec((1, 1, _TM), lambda i, k: (i, 0, 0)),
            pl.BlockSpec((1, 1), lambda i, k: (0, 0)),
        ],
        out_shape=[
            jax.ShapeDtypeStruct((ni, 1, _TM), jnp.int32),
            jax.ShapeDtypeStruct((1, 1), jnp.float32),
        ],
        scratch_shapes=[
            pltpu.VMEM((8, _TM), jnp.float32),
            pltpu.VMEM((8, _TM), jnp.float32),
            pltpu.VMEM((1, _TM), jnp.float32),
            pltpu.VMEM((nk * _TK, 1), jnp.float32),
        ],
        compiler_params=pltpu.CompilerParams(
            dimension_semantics=("arbitrary", "arbitrary")),
    )(zf, w)
    return idx3.reshape(m), loss[0, 0]


def _gather_body(w_hbm, idx_hbm, out_hbm, idx_v, rows_v, sem):
    wid = lax.axis_index("s") * _NC + lax.axis_index("c")
    b_per_w = idx_hbm.shape[0] // _NW
    base = wid * b_per_w
    nch = b_per_w // _CHUNK

    def chunk(c, carry):
        off = base + c * _CHUNK
        pltpu.sync_copy(idx_hbm.at[pl.ds(off, _CHUNK)], idx_v)
        pltpu.async_copy(w_hbm.at[idx_v], rows_v, sem).wait()
        pltpu.sync_copy(rows_v, out_hbm.at[pl.ds(off, _CHUNK)])
        return carry

    lax.fori_loop(0, nch, chunk, 0)


def _vq_gather(w, idx):
    m = idx.shape[0]
    d_dim = w.shape[1]
    mesh = plsc.VectorSubcoreMesh(core_axis_name="c", subcore_axis_name="s")
    fn = functools.partial(
        pl.kernel,
        mesh=mesh,
        out_type=jax.ShapeDtypeStruct((m, d_dim), jnp.float32),
        scratch_types=[
            pltpu.VMEM((_CHUNK,), jnp.int32),
            pltpu.VMEM((_CHUNK, d_dim), jnp.float32),
            pltpu.SemaphoreType.DMA,
        ],
    )(_gather_body)
    return fn(w, idx)


def kernel(z, W):
    zf = z.reshape(-1, z.shape[-1])
    idx, loss = _vq_argmin(zf, W)
    z_q = _vq_gather(W, idx)
    z_q_st = z_q.reshape(z.shape)
    min_encoding_indices = idx.reshape(z.shape[:-1] + (1,))
    return (z_q_st, loss, min_encoding_indices)
